# baseline (speedup 1.0000x reference)
import jax
import jax.numpy as jnp
from jax import lax
from jax.experimental import pallas as pl
from jax.experimental.pallas import tpu as pltpu
from jax.experimental.pallas import tpu_sc as plsc

B = 1024
S = 512
EMB = 128
CHUNK = 128
LANES = 16
NW = 32
NBUF = 4
CHUNKS_PER_W = (B * S) // (CHUNK * NW)
SEQ_PER_W = 64


def _emb_kernel(table_hbm, idxarr_hbm, pos_hbm, out_hbm,
                pos_v, idx_v, rows0, rows1, rows2, rows3,
                g0, g1, g2, g3, o0, o1, o2, o3):
    cid = lax.axis_index("c")
    sid = lax.axis_index("s")
    wid = sid * 2 + cid
    half = cid
    bgrp = sid

    rows = [rows0, rows1, rows2, rows3]
    gsem = [g0, g1, g2, g3]
    osem = [o0, o1, o2, o3]

    pltpu.sync_copy(pos_hbm.at[pl.ds(half * 256, 256)], pos_v)
    pltpu.sync_copy(idxarr_hbm.at[wid], idx_v)

    seq0 = bgrp * SEQ_PER_W

    def fire_gather(i, j, bb):
        pltpu.make_async_copy(
            table_hbm.at[idx_v.at[i, j, pl.ds(0, 64)]],
            rows[bb].at[pl.ds(0, 64)], gsem[bb]).start()
        pltpu.make_async_copy(
            table_hbm.at[idx_v.at[i, j, pl.ds(64, 64)]],
            rows[bb].at[pl.ds(64, 64)], gsem[bb]).start()

    fire_gather(0, 0, 0)
    fire_gather(0, 1, 1)

    def outer(g, carry):
        for bb in range(NBUF):
            j = bb % 2
            k = g * NBUF + bb

            nb = (bb + 2) % NBUF

            @pl.when(k + 2 < CHUNKS_PER_W)
            def _():
                @pl.when(k >= 2)
                def _():
                    pltpu.make_async_copy(
                        rows[nb], out_hbm.at[pl.ds(0, CHUNK)], osem[nb]
                    ).wait()
                fire_gather((k + 2) // 2, j, nb)

            pltpu.make_async_copy(
                table_hbm.at[idx_v.at[0, 0]], rows[bb], gsem[bb]).wait()

            poff = j * CHUNK

            rbuf = rows[bb]

            @plsc.parallel_loop(0, CHUNK, step=1, unroll=4)
            def _(t):
                for v in range(EMB // LANES):
                    sl = pl.ds(v * LANES, LANES)
                    plsc.addupdate(rbuf.at[t, sl], pos_v[poff + t, sl])

            i = g * 2 + bb // 2
            tok0 = (seq0 + i) * S + half * 256 + j * CHUNK
            pltpu.make_async_copy(
                rows[bb], out_hbm.at[pl.ds(tok0, CHUNK)], osem[bb]).start()
        return carry

    lax.fori_loop(0, CHUNKS_PER_W // NBUF, outer, 0)

    for bb in range(NBUF):
        pltpu.make_async_copy(
            rows[bb], out_hbm.at[pl.ds(0, CHUNK)], osem[bb]).wait()


@jax.jit
def _emb(table, idxarr, pos):
    mesh = plsc.VectorSubcoreMesh(core_axis_name="c", subcore_axis_name="s")
    return pl.kernel(
        _emb_kernel,
        mesh=mesh,
        out_type=jax.ShapeDtypeStruct((B * S, EMB), jnp.float32),
        scratch_types=[
            pltpu.VMEM((256, EMB), jnp.float32),
            pltpu.VMEM((SEQ_PER_W, 2, 128), jnp.int32),
            pltpu.VMEM((CHUNK, EMB), jnp.float32),
            pltpu.VMEM((CHUNK, EMB), jnp.float32),
            pltpu.VMEM((CHUNK, EMB), jnp.float32),
            pltpu.VMEM((CHUNK, EMB), jnp.float32),
            pltpu.SemaphoreType.DMA,
            pltpu.SemaphoreType.DMA,
            pltpu.SemaphoreType.DMA,
            pltpu.SemaphoreType.DMA,
            pltpu.SemaphoreType.DMA,
            pltpu.SemaphoreType.DMA,
            pltpu.SemaphoreType.DMA,
            pltpu.SemaphoreType.DMA,
        ],
    )(table, idxarr, pos)


def kernel(inputs, token_table, position_embedding):
    idxarr = (inputs.astype(jnp.int32)
              .reshape(16, 64, 2, 2, 128)
              .transpose(0, 2, 1, 3, 4)
              .reshape(NW, SEQ_PER_W, 2, 128))
    out = _emb(token_table, idxarr, position_embedding)
    return out.reshape(B, S, EMB)


# speedup vs baseline: 1.0076x; 1.0076x over previous
import jax
import jax.numpy as jnp
from jax import lax
from jax.experimental import pallas as pl
from jax.experimental.pallas import tpu as pltpu
from jax.experimental.pallas import tpu_sc as plsc

B = 1024
S = 512
EMB = 128
CHUNK = 128
LANES = 16
NW = 32
NBUF = 4
CHUNKS_PER_W = (B * S) // (CHUNK * NW)
SEQ_PER_W = 64


def _emb_kernel(table_hbm, idxarr_hbm, pos_hbm, out_hbm,
                pos_v, idx_v, rows0, rows1, rows2, rows3,
                g0, g1, g2, g3, o0, o1, o2, o3):
    cid = lax.axis_index("c")
    sid = lax.axis_index("s")
    wid = sid * 2 + cid
    half = cid
    bgrp = sid

    rows = [rows0, rows1, rows2, rows3]
    gsem = [g0, g1, g2, g3]
    osem = [o0, o1, o2, o3]

    pltpu.sync_copy(pos_hbm.at[pl.ds(half * 256, 256)], pos_v)
    pltpu.sync_copy(
        idxarr_hbm.at[pl.ds(bgrp * SEQ_PER_W, SEQ_PER_W),
                      pl.ds(half * 256, 256)], idx_v)

    seq0 = bgrp * SEQ_PER_W

    def fire_gather(i, j, bb):
        pltpu.make_async_copy(
            table_hbm.at[idx_v.at[i, pl.ds(j * CHUNK, CHUNK)]],
            rows[bb], gsem[bb]).start()

    fire_gather(0, 0, 0)
    fire_gather(0, 1, 1)

    def outer(g, carry):
        for bb in range(NBUF):
            j = bb % 2
            k = g * NBUF + bb

            nb = (bb + 2) % NBUF

            @pl.when(k + 2 < CHUNKS_PER_W)
            def _():
                @pl.when(k >= 2)
                def _():
                    pltpu.make_async_copy(
                        rows[nb], out_hbm.at[pl.ds(0, CHUNK)], osem[nb]
                    ).wait()
                fire_gather((k + 2) // 2, j, nb)

            pltpu.make_async_copy(
                table_hbm.at[idx_v.at[0, pl.ds(0, CHUNK)]],
                rows[bb], gsem[bb]).wait()

            poff = j * CHUNK

            rbuf = rows[bb]

            @plsc.parallel_loop(0, CHUNK, step=1, unroll=4)
            def _(t):
                for v in range(EMB // LANES):
                    sl = pl.ds(v * LANES, LANES)
                    plsc.addupdate(rbuf.at[t, sl], pos_v[poff + t, sl])

            i = g * 2 + bb // 2
            tok0 = (seq0 + i) * S + half * 256 + j * CHUNK
            pltpu.make_async_copy(
                rows[bb], out_hbm.at[pl.ds(tok0, CHUNK)], osem[bb]).start()
        return carry

    lax.fori_loop(0, CHUNKS_PER_W // NBUF, outer, 0)

    for bb in range(NBUF):
        pltpu.make_async_copy(
            rows[bb], out_hbm.at[pl.ds(0, CHUNK)], osem[bb]).wait()


@jax.jit
def _emb(table, idxarr, pos):
    mesh = plsc.VectorSubcoreMesh(core_axis_name="c", subcore_axis_name="s")
    return pl.kernel(
        _emb_kernel,
        mesh=mesh,
        out_type=jax.ShapeDtypeStruct((B * S, EMB), jnp.float32),
        scratch_types=[
            pltpu.VMEM((256, EMB), jnp.float32),
            pltpu.VMEM((SEQ_PER_W, 256), jnp.int32),
            pltpu.VMEM((CHUNK, EMB), jnp.float32),
            pltpu.VMEM((CHUNK, EMB), jnp.float32),
            pltpu.VMEM((CHUNK, EMB), jnp.float32),
            pltpu.VMEM((CHUNK, EMB), jnp.float32),
            pltpu.SemaphoreType.DMA,
            pltpu.SemaphoreType.DMA,
            pltpu.SemaphoreType.DMA,
            pltpu.SemaphoreType.DMA,
            pltpu.SemaphoreType.DMA,
            pltpu.SemaphoreType.DMA,
            pltpu.SemaphoreType.DMA,
            pltpu.SemaphoreType.DMA,
        ],
    )(table, idxarr, pos)


def kernel(inputs, token_table, position_embedding):
    out = _emb(token_table, inputs.astype(jnp.int32), position_embedding)
    return out.reshape(B, S, EMB)


# speedup vs baseline: 1.0100x; 1.0024x over previous
import jax
import jax.numpy as jnp
from jax import lax
from jax.experimental import pallas as pl
from jax.experimental.pallas import tpu as pltpu
from jax.experimental.pallas import tpu_sc as plsc

B = 1024
S = 512
EMB = 128
CHUNK = 128
LANES = 16
NW = 32
NBUF = 5
DEPTH = 3
CHUNKS_PER_W = (B * S) // (CHUNK * NW)
SEQ_PER_GRP = 128


def _emb_kernel(table_hbm, idxarr_hbm, pos_hbm, out_hbm,
                pos_v, idx_v, rows0, rows1, rows2, rows3, rows4,
                g0, g1, g2, g3, g4, o0, o1, o2, o3, o4):
    cid = lax.axis_index("c")
    sid = lax.axis_index("s")
    wid = sid * 2 + cid
    quarter = wid % 4
    grp = wid // 4

    rows = [rows0, rows1, rows2, rows3, rows4]
    gsem = [g0, g1, g2, g3, g4]
    osem = [o0, o1, o2, o3, o4]

    pltpu.sync_copy(pos_hbm.at[pl.ds(quarter * CHUNK, CHUNK)], pos_v)
    pltpu.sync_copy(
        idxarr_hbm.at[pl.ds(grp * SEQ_PER_GRP, SEQ_PER_GRP),
                      pl.ds(quarter * CHUNK, CHUNK)], idx_v)

    seq0 = grp * SEQ_PER_GRP

    def fire_gather(k, bb):
        pltpu.make_async_copy(
            table_hbm.at[idx_v.at[k]], rows[bb], gsem[bb]).start()

    for k0 in range(DEPTH):
        fire_gather(k0, k0)

    def outer(g, carry):
        for bb in range(NBUF):
            k = g * NBUF + bb

            nb = (bb + DEPTH) % NBUF

            @pl.when(k + DEPTH < CHUNKS_PER_W)
            def _():
                @pl.when(k >= 2)
                def _():
                    pltpu.make_async_copy(
                        rows[nb], out_hbm.at[pl.ds(0, CHUNK)], osem[nb]
                    ).wait()
                fire_gather(k + DEPTH, nb)

            @pl.when(k < CHUNKS_PER_W)
            def _():
                pltpu.make_async_copy(
                    table_hbm.at[idx_v.at[0]], rows[bb], gsem[bb]).wait()

                rbuf = rows[bb]

                @plsc.parallel_loop(0, CHUNK, step=1, unroll=4)
                def _(t):
                    for v in range(EMB // LANES):
                        sl = pl.ds(v * LANES, LANES)
                        plsc.addupdate(rbuf.at[t, sl], pos_v[t, sl])

                tok0 = (seq0 + k) * S + quarter * CHUNK
                pltpu.make_async_copy(
                    rbuf, out_hbm.at[pl.ds(tok0, CHUNK)], osem[bb]).start()
        return carry

    n_outer = (CHUNKS_PER_W + NBUF - 1) // NBUF
    lax.fori_loop(0, n_outer, outer, 0)

    for bb in range(NBUF):
        pltpu.make_async_copy(
            rows[bb], out_hbm.at[pl.ds(0, CHUNK)], osem[bb]).wait()


@jax.jit
def _emb(table, idxarr, pos):
    mesh = plsc.VectorSubcoreMesh(core_axis_name="c", subcore_axis_name="s")
    return pl.kernel(
        _emb_kernel,
        mesh=mesh,
        out_type=jax.ShapeDtypeStruct((B * S, EMB), jnp.float32),
        scratch_types=[
            pltpu.VMEM((CHUNK, EMB), jnp.float32),
            pltpu.VMEM((SEQ_PER_GRP, CHUNK), jnp.int32),
            pltpu.VMEM((CHUNK, EMB), jnp.float32),
            pltpu.VMEM((CHUNK, EMB), jnp.float32),
            pltpu.VMEM((CHUNK, EMB), jnp.float32),
            pltpu.VMEM((CHUNK, EMB), jnp.float32),
            pltpu.VMEM((CHUNK, EMB), jnp.float32),
            pltpu.SemaphoreType.DMA,
            pltpu.SemaphoreType.DMA,
            pltpu.SemaphoreType.DMA,
            pltpu.SemaphoreType.DMA,
            pltpu.SemaphoreType.DMA,
            pltpu.SemaphoreType.DMA,
            pltpu.SemaphoreType.DMA,
            pltpu.SemaphoreType.DMA,
            pltpu.SemaphoreType.DMA,
            pltpu.SemaphoreType.DMA,
        ],
    )(table, idxarr, pos)


def kernel(inputs, token_table, position_embedding):
    out = _emb(token_table, inputs.astype(jnp.int32), position_embedding)
    return out.reshape(B, S, EMB)
